# baseline (speedup 1.0000x reference)
import functools

import jax
import jax.numpy as jnp
from jax import lax
from jax.experimental import pallas as pl
from jax.experimental.pallas import tpu as pltpu
from jax.experimental.pallas import tpu_sc as plsc

N = 50000
E = 800000
G = 512
H = 128
F_IN = 78

NC, NS, L = 2, 16, 16
NW = NC * NS

CHUNK = 12544
NPAD = 4 * CHUNK
DUMP = CHUNK
ACC_ROWS = 12800
ZROWS = 800

EPAD = 819200
EB = 128
SEB = 2048
EDGES_PER_TILE = EPAD // NS
N_STAGES = EDGES_PER_TILE // SEB
BLKS_PER_STAGE = SEB // EB
WB_ROWS = CHUNK // NS

SROWS = 640
PNB = 112
NODES_PER_TILE = NPAD // NW
PN_BLKS = NODES_PER_TILE // PNB

_mesh = plsc.VectorSubcoreMesh(
    core_axis_name="c", subcore_axis_name="s", num_cores=NC, num_subcores=NS)


@functools.partial(
    pl.kernel,
    out_type=jax.ShapeDtypeStruct((NPAD, H), jnp.float32),
    mesh=_mesh,
    scratch_types=[
        pltpu.VMEM((SEB,), jnp.int32),
        pltpu.VMEM((SEB,), jnp.int32),
        pltpu.VMEM((EB,), jnp.int32),
        pltpu.VMEM((EB,), jnp.int32),
        pltpu.VMEM((EB, H), jnp.float32),
        pltpu.VMEM_SHARED((ACC_ROWS, H), jnp.float32),
        pltpu.SemaphoreType.DMA,
    ],
)
def _sc_agg(h_hbm, src_hbm, dst_hbm, zeros_hbm, out_hbm,
            sstage, dstage, idxb, locb, rows, acc, sem):
    cid = lax.axis_index("c")
    sid = lax.axis_index("s")
    for p in range(2):
        cc = cid * 2 + p
        lo = cc * CHUNK
        pltpu.sync_copy(zeros_hbm.at[pl.ds(0, ZROWS)],
                        acc.at[pl.ds(sid * ZROWS, ZROWS)])
        plsc.subcore_barrier()

        def stage_body(st, _):
            sbase = sid * EDGES_PER_TILE + st * SEB
            pltpu.sync_copy(src_hbm.at[pl.ds(sbase, SEB)], sstage)
            pltpu.sync_copy(dst_hbm.at[pl.ds(sbase, SEB)], dstage)

            def blk_body(b, _):
                for k in range(EB // L):
                    off = b * EB + k * L
                    s16 = sstage[pl.ds(off, L)]
                    d16 = dstage[pl.ds(off, L)]
                    m = (d16 >= lo) & (d16 < lo + CHUNK)
                    idxb[pl.ds(k * L, L)] = s16
                    locb[pl.ds(k * L, L)] = jnp.where(m, d16 - lo, DUMP)
                pltpu.async_copy(h_hbm.at[idxb], rows, sem).wait()
                pltpu.sync_copy(rows, acc.at[locb], add=True)
                return 0

            lax.fori_loop(0, BLKS_PER_STAGE, blk_body, 0)
            return 0

        lax.fori_loop(0, N_STAGES, stage_body, 0)
        plsc.subcore_barrier()
        pltpu.sync_copy(acc.at[pl.ds(sid * WB_ROWS, WB_ROWS)],
                        out_hbm.at[pl.ds(lo + sid * WB_ROWS, WB_ROWS)])
        plsc.subcore_barrier()


@functools.partial(
    pl.kernel,
    out_type=(
        jax.ShapeDtypeStruct((NC * SROWS, H), jnp.float32),
        jax.ShapeDtypeStruct((NC * SROWS, H), jnp.float32),
        jax.ShapeDtypeStruct((NW * SROWS, H), jnp.float32),
    ),
    mesh=_mesh,
    scratch_types=[
        pltpu.VMEM((PNB, H), jnp.float32),
        pltpu.VMEM((PNB,), jnp.int32),
        pltpu.VMEM((PNB + L,), jnp.int32),
        pltpu.VMEM((PNB, H), jnp.float32),
        pltpu.VMEM((SROWS, H), jnp.float32),
        pltpu.VMEM_SHARED((SROWS, H), jnp.float32),
        pltpu.VMEM_SHARED((SROWS, H), jnp.float32),
    ],
)
def _sc_pool(h_hbm, batch_hbm, zeros_hbm, ones_hbm,
             sums_hbm, cnts_hbm, maxp_hbm,
             rows, bstage, bpad, ones_v, maxacc, sumacc, cntacc):
    cid = lax.axis_index("c")
    sid = lax.axis_index("s")
    w = cid * NS + sid
    zr = SROWS // NS
    pltpu.sync_copy(zeros_hbm.at[pl.ds(0, SROWS)], maxacc)
    pltpu.sync_copy(ones_hbm, ones_v)
    pltpu.sync_copy(zeros_hbm.at[pl.ds(0, zr)], sumacc.at[pl.ds(sid * zr, zr)])
    pltpu.sync_copy(zeros_hbm.at[pl.ds(0, zr)], cntacc.at[pl.ds(sid * zr, zr)])
    plsc.subcore_barrier()

    for t in range(PN_BLKS):
        nbase = w * NODES_PER_TILE + t * PNB
        pltpu.sync_copy(h_hbm.at[pl.ds(nbase, PNB)], rows)
        pltpu.sync_copy(batch_hbm.at[pl.ds(nbase, PNB)], bstage)
        pltpu.sync_copy(batch_hbm.at[pl.ds(nbase, PNB)], bpad.at[pl.ds(0, PNB)])
        pltpu.sync_copy(rows, sumacc.at[bstage], add=True)
        pltpu.sync_copy(ones_v, cntacc.at[bstage], add=True)

        def nbody(i, _):
            b = bpad[pl.ds(i, L)][0]
            for j in range(H // L):
                sl = pl.ds(j * L, L)
                maxacc[b, sl] = jnp.maximum(maxacc[b, sl], rows[i, sl])
            return 0

        lax.fori_loop(0, PNB, nbody, 0)

    pltpu.sync_copy(maxacc, maxp_hbm.at[pl.ds(w * SROWS, SROWS)])
    plsc.subcore_barrier()
    pltpu.sync_copy(sumacc.at[pl.ds(sid * zr, zr)],
                    sums_hbm.at[pl.ds(cid * SROWS + sid * zr, zr)])
    pltpu.sync_copy(cntacc.at[pl.ds(sid * zr, zr)],
                    cnts_hbm.at[pl.ds(cid * SROWS + sid * zr, zr)])


BM = 512


def _mlp_body(h_ref, a_ref, w1_ref, b1_ref, w2_ref, b2_ref, o_ref):
    z = h_ref[...] + a_ref[...]
    t = jnp.maximum(
        jnp.dot(z, w1_ref[...], preferred_element_type=jnp.float32)
        + b1_ref[...], 0.0)
    o_ref[...] = jnp.maximum(
        jnp.dot(t, w2_ref[...], preferred_element_type=jnp.float32)
        + b2_ref[...], 0.0)


def _tc_mlp(h, agg, w1, b1, w2, b2):
    return pl.pallas_call(
        _mlp_body,
        grid=(NPAD // BM,),
        in_specs=[
            pl.BlockSpec((BM, H), lambda i: (i, 0)),
            pl.BlockSpec((BM, H), lambda i: (i, 0)),
            pl.BlockSpec((H, H), lambda i: (0, 0)),
            pl.BlockSpec((1, H), lambda i: (0, 0)),
            pl.BlockSpec((H, H), lambda i: (0, 0)),
            pl.BlockSpec((1, H), lambda i: (0, 0)),
        ],
        out_specs=pl.BlockSpec((BM, H), lambda i: (i, 0)),
        out_shape=jax.ShapeDtypeStruct((NPAD, H), jnp.float32),
    )(h, agg, w1, b1.reshape(1, H), w2, b2.reshape(1, H))


def _final_body(sum_ref, cnt_ref, maxp_ref, wf_ref, bf_ref, o_ref):
    s = sum_ref[0, :G, :] + sum_ref[1, :G, :]
    c = cnt_ref[0, :G, 0:1] + cnt_ref[1, :G, 0:1]
    mean = s / jnp.clip(c, 1.0, None)
    mx = jnp.max(maxp_ref[:, :G, :], axis=0)
    o = (jnp.dot(mean, wf_ref[:H, :], preferred_element_type=jnp.float32)
         + jnp.dot(mx, wf_ref[H:, :], preferred_element_type=jnp.float32)
         + bf_ref[...])
    o_ref[...] = o


def _tc_final(sums, cnts, maxp, wf, bf):
    return pl.pallas_call(
        _final_body,
        out_shape=jax.ShapeDtypeStruct((G, H), jnp.float32),
    )(sums, cnts, maxp, wf, bf.reshape(1, H))


def kernel(x, edge_index, batch,
           W1_0, b1_0, W2_0, b2_0,
           W1_1, b1_1, W2_1, b2_1,
           W1_2, b1_2, W2_2, b2_2,
           Wf, bf):
    src = jnp.pad(edge_index[0], (0, EPAD - E))
    dst = jnp.pad(edge_index[1], (0, EPAD - E), constant_values=N)
    xp = jnp.pad(x, ((0, NPAD - N), (0, H - F_IN)))
    w1_0p = jnp.pad(W1_0, ((0, H - F_IN), (0, 0)))
    batchp = jnp.pad(batch, (0, NPAD - N), constant_values=G)

    zeros = jnp.zeros((ACC_ROWS, H), jnp.float32)
    ones = jnp.ones((PNB, H), jnp.float32)

    h = xp
    for w1, b1, w2, b2 in ((w1_0p, b1_0, W2_0, b2_0),
                           (W1_1, b1_1, W2_1, b2_1),
                           (W1_2, b1_2, W2_2, b2_2)):
        agg = _sc_agg(h, src, dst, zeros)
        h = _tc_mlp(h, agg, w1, b1, w2, b2)

    sums, cnts, maxp = _sc_pool(h, batchp, zeros, ones)
    return _tc_final(sums.reshape(NC, SROWS, H), cnts.reshape(NC, SROWS, H),
                     maxp.reshape(NW, SROWS, H), Wf, bf)


# speedup vs baseline: 1.0097x; 1.0097x over previous
import functools

import jax
import jax.numpy as jnp
from jax import lax
from jax.experimental import pallas as pl
from jax.experimental.pallas import tpu as pltpu
from jax.experimental.pallas import tpu_sc as plsc

N = 50000
E = 800000
G = 512
H = 128
F_IN = 78

NC, NS, L = 2, 16, 16
NW = NC * NS

CHUNK = 12544
NPAD = 4 * CHUNK
DUMP = CHUNK
ACC_ROWS = 12672
ZROWS = 792

EPAD = 819200
EB = 64
SEB = 2048
EDGES_PER_TILE = EPAD // NS
N_STAGES = EDGES_PER_TILE // SEB
BLKS_PER_STAGE = SEB // EB
WB_ROWS = CHUNK // NS

SROWS = 640
PNB = 112
NODES_PER_TILE = NPAD // NW
PN_BLKS = NODES_PER_TILE // PNB

_mesh = plsc.VectorSubcoreMesh(
    core_axis_name="c", subcore_axis_name="s", num_cores=NC, num_subcores=NS)


@functools.partial(
    pl.kernel,
    out_type=jax.ShapeDtypeStruct((NPAD, H), jnp.float32),
    mesh=_mesh,
    scratch_types=[
        pltpu.VMEM((SEB,), jnp.int32),
        pltpu.VMEM((SEB,), jnp.int32),
        pltpu.VMEM((EB,), jnp.int32),
        pltpu.VMEM((EB,), jnp.int32),
        pltpu.VMEM((EB,), jnp.int32),
        pltpu.VMEM((EB,), jnp.int32),
        pltpu.VMEM((EB, H), jnp.float32),
        pltpu.VMEM((EB, H), jnp.float32),
        pltpu.VMEM_SHARED((ACC_ROWS, H), jnp.float32),
        pltpu.SemaphoreType.DMA,
        pltpu.SemaphoreType.DMA,
        pltpu.SemaphoreType.DMA,
        pltpu.SemaphoreType.DMA,
    ],
)
def _sc_agg(h_hbm, src_hbm, dst_hbm, zeros_hbm, out_hbm,
            sstage, dstage, idx0, idx1, loc0, loc1, rows0, rows1, acc,
            gsem0, gsem1, ssem0, ssem1):
    cid = lax.axis_index("c")
    sid = lax.axis_index("s")
    idxs = (idx0, idx1)
    locs = (loc0, loc1)
    rows = (rows0, rows1)
    gsems = (gsem0, gsem1)
    ssems = (ssem0, ssem1)
    nblk = EDGES_PER_TILE // EB

    def load_stage(st):
        sbase = sid * EDGES_PER_TILE + st * SEB
        pltpu.sync_copy(src_hbm.at[pl.ds(sbase, SEB)], sstage)
        pltpu.sync_copy(dst_hbm.at[pl.ds(sbase, SEB)], dstage)

    for p in range(2):
        cc = cid * 2 + p
        lo = cc * CHUNK
        pltpu.sync_copy(zeros_hbm.at[pl.ds(0, ZROWS)],
                        acc.at[pl.ds(sid * ZROWS, ZROWS)])
        plsc.subcore_barrier()

        def build(slot, b):
            boff = lax.rem(b, BLKS_PER_STAGE) * EB
            for k in range(EB // L):
                off = boff + k * L
                s16 = sstage[pl.ds(off, L)]
                d16 = dstage[pl.ds(off, L)]
                m = (d16 >= lo) & (d16 < lo + CHUNK)
                idxs[slot][pl.ds(k * L, L)] = s16
                locs[slot][pl.ds(k * L, L)] = jnp.where(m, d16 - lo, DUMP)

        def gather(slot):
            pltpu.async_copy(h_hbm.at[idxs[slot]], rows[slot], gsems[slot])

        def wait_gather(slot):
            pltpu.make_async_copy(h_hbm.at[idxs[slot]], rows[slot],
                                  gsems[slot]).wait()

        def scatter(slot):
            pltpu.async_copy(rows[slot], acc.at[locs[slot]], ssems[slot],
                             add=True)

        def wait_scatter(slot):
            pltpu.make_async_copy(rows[slot], acc.at[locs[slot]],
                                  ssems[slot]).wait()

        load_stage(0)
        build(0, 0)
        gather(0)

        def group(g, _):
            b0 = 2 * g
            wait_gather(0)
            scatter(0)
            @pl.when(g > 0)
            def _():
                wait_scatter(1)
            build(1, b0 + 1)
            gather(1)
            wait_gather(1)
            scatter(1)
            @pl.when(g < nblk // 2 - 1)
            def _():
                wait_scatter(0)
                @pl.when(lax.rem(g, BLKS_PER_STAGE // 2)
                         == BLKS_PER_STAGE // 2 - 1)
                def _():
                    load_stage((g + 1) // (BLKS_PER_STAGE // 2))
                build(0, b0 + 2)
                gather(0)
            return 0

        lax.fori_loop(0, nblk // 2, group, 0)
        wait_scatter(0)
        wait_scatter(1)
        plsc.subcore_barrier()
        pltpu.sync_copy(acc.at[pl.ds(sid * WB_ROWS, WB_ROWS)],
                        out_hbm.at[pl.ds(lo + sid * WB_ROWS, WB_ROWS)])
        plsc.subcore_barrier()


@functools.partial(
    pl.kernel,
    out_type=(
        jax.ShapeDtypeStruct((NC * SROWS, H), jnp.float32),
        jax.ShapeDtypeStruct((NC * SROWS, H), jnp.float32),
        jax.ShapeDtypeStruct((NW * SROWS, H), jnp.float32),
    ),
    mesh=_mesh,
    scratch_types=[
        pltpu.VMEM((PNB, H), jnp.float32),
        pltpu.VMEM((PNB,), jnp.int32),
        pltpu.VMEM((PNB + L,), jnp.int32),
        pltpu.VMEM((PNB, H), jnp.float32),
        pltpu.VMEM((SROWS, H), jnp.float32),
        pltpu.VMEM_SHARED((SROWS, H), jnp.float32),
        pltpu.VMEM_SHARED((SROWS, H), jnp.float32),
    ],
)
def _sc_pool(h_hbm, batch_hbm, zeros_hbm, ones_hbm,
             sums_hbm, cnts_hbm, maxp_hbm,
             rows, bstage, bpad, ones_v, maxacc, sumacc, cntacc):
    cid = lax.axis_index("c")
    sid = lax.axis_index("s")
    w = cid * NS + sid
    zr = SROWS // NS
    pltpu.sync_copy(zeros_hbm.at[pl.ds(0, SROWS)], maxacc)
    pltpu.sync_copy(ones_hbm, ones_v)
    pltpu.sync_copy(zeros_hbm.at[pl.ds(0, zr)], sumacc.at[pl.ds(sid * zr, zr)])
    pltpu.sync_copy(zeros_hbm.at[pl.ds(0, zr)], cntacc.at[pl.ds(sid * zr, zr)])
    plsc.subcore_barrier()

    for t in range(PN_BLKS):
        nbase = w * NODES_PER_TILE + t * PNB
        pltpu.sync_copy(h_hbm.at[pl.ds(nbase, PNB)], rows)
        pltpu.sync_copy(batch_hbm.at[pl.ds(nbase, PNB)], bstage)
        pltpu.sync_copy(batch_hbm.at[pl.ds(nbase, PNB)], bpad.at[pl.ds(0, PNB)])
        pltpu.sync_copy(rows, sumacc.at[bstage], add=True)
        pltpu.sync_copy(ones_v, cntacc.at[bstage], add=True)

        def nbody(i, _):
            b = bpad[pl.ds(i, L)][0]
            for j in range(H // L):
                sl = pl.ds(j * L, L)
                maxacc[b, sl] = jnp.maximum(maxacc[b, sl], rows[i, sl])
            return 0

        lax.fori_loop(0, PNB, nbody, 0)

    pltpu.sync_copy(maxacc, maxp_hbm.at[pl.ds(w * SROWS, SROWS)])
    plsc.subcore_barrier()
    pltpu.sync_copy(sumacc.at[pl.ds(sid * zr, zr)],
                    sums_hbm.at[pl.ds(cid * SROWS + sid * zr, zr)])
    pltpu.sync_copy(cntacc.at[pl.ds(sid * zr, zr)],
                    cnts_hbm.at[pl.ds(cid * SROWS + sid * zr, zr)])


BM = 512


def _mlp_body(h_ref, a_ref, w1_ref, b1_ref, w2_ref, b2_ref, o_ref):
    z = h_ref[...] + a_ref[...]
    t = jnp.maximum(
        jnp.dot(z, w1_ref[...], preferred_element_type=jnp.float32)
        + b1_ref[...], 0.0)
    o_ref[...] = jnp.maximum(
        jnp.dot(t, w2_ref[...], preferred_element_type=jnp.float32)
        + b2_ref[...], 0.0)


def _tc_mlp(h, agg, w1, b1, w2, b2):
    return pl.pallas_call(
        _mlp_body,
        grid=(NPAD // BM,),
        in_specs=[
            pl.BlockSpec((BM, H), lambda i: (i, 0)),
            pl.BlockSpec((BM, H), lambda i: (i, 0)),
            pl.BlockSpec((H, H), lambda i: (0, 0)),
            pl.BlockSpec((1, H), lambda i: (0, 0)),
            pl.BlockSpec((H, H), lambda i: (0, 0)),
            pl.BlockSpec((1, H), lambda i: (0, 0)),
        ],
        out_specs=pl.BlockSpec((BM, H), lambda i: (i, 0)),
        out_shape=jax.ShapeDtypeStruct((NPAD, H), jnp.float32),
    )(h, agg, w1, b1.reshape(1, H), w2, b2.reshape(1, H))


def _final_body(sum_ref, cnt_ref, maxp_ref, wf_ref, bf_ref, o_ref):
    s = sum_ref[0, :G, :] + sum_ref[1, :G, :]
    c = cnt_ref[0, :G, 0:1] + cnt_ref[1, :G, 0:1]
    mean = s / jnp.clip(c, 1.0, None)
    mx = jnp.max(maxp_ref[:, :G, :], axis=0)
    o = (jnp.dot(mean, wf_ref[:H, :], preferred_element_type=jnp.float32)
         + jnp.dot(mx, wf_ref[H:, :], preferred_element_type=jnp.float32)
         + bf_ref[...])
    o_ref[...] = o


def _tc_final(sums, cnts, maxp, wf, bf):
    return pl.pallas_call(
        _final_body,
        out_shape=jax.ShapeDtypeStruct((G, H), jnp.float32),
    )(sums, cnts, maxp, wf, bf.reshape(1, H))


def kernel(x, edge_index, batch,
           W1_0, b1_0, W2_0, b2_0,
           W1_1, b1_1, W2_1, b2_1,
           W1_2, b1_2, W2_2, b2_2,
           Wf, bf):
    src = jnp.pad(edge_index[0], (0, EPAD - E))
    dst = jnp.pad(edge_index[1], (0, EPAD - E), constant_values=N)
    xp = jnp.pad(x, ((0, NPAD - N), (0, H - F_IN)))
    w1_0p = jnp.pad(W1_0, ((0, H - F_IN), (0, 0)))
    batchp = jnp.pad(batch, (0, NPAD - N), constant_values=G)

    zeros = jnp.zeros((ACC_ROWS, H), jnp.float32)
    ones = jnp.ones((PNB, H), jnp.float32)

    h = xp
    for w1, b1, w2, b2 in ((w1_0p, b1_0, W2_0, b2_0),
                           (W1_1, b1_1, W2_1, b2_1),
                           (W1_2, b1_2, W2_2, b2_2)):
        agg = _sc_agg(h, src, dst, zeros)
        h = _tc_mlp(h, agg, w1, b1, w2, b2)

    sums, cnts, maxp = _sc_pool(h, batchp, zeros, ones)
    return _tc_final(sums.reshape(NC, SROWS, H), cnts.reshape(NC, SROWS, H),
                     maxp.reshape(NW, SROWS, H), Wf, bf)


# speedup vs baseline: 1.4729x; 1.4588x over previous
import functools

import jax
import jax.numpy as jnp
from jax import lax
from jax.experimental import pallas as pl
from jax.experimental.pallas import tpu as pltpu
from jax.experimental.pallas import tpu_sc as plsc

N = 50000
E = 800000
G = 512
H = 128
F_IN = 78

NC, NS, L = 2, 16, 16
NW = NC * NS

CHUNK = 12544
NPAD = 4 * CHUNK
DUMP = CHUNK
ACC_ROWS = 12672
ZROWS = 792

EPAD = 819200
EB = 64
SEB = 2048
EDGES_PER_TILE = EPAD // NS
N_STAGES = EDGES_PER_TILE // SEB
BLKS_PER_STAGE = SEB // EB
WB_ROWS = CHUNK // NS

SROWS = 640
PNB = 112
NODES_PER_TILE = NPAD // NW
PN_BLKS = NODES_PER_TILE // PNB

_mesh = plsc.VectorSubcoreMesh(
    core_axis_name="c", subcore_axis_name="s", num_cores=NC, num_subcores=NS)


SSTG = 576
BSEB = 1600
CAP = 25664


@functools.partial(
    pl.kernel,
    out_type=(
        jax.ShapeDtypeStruct((NW * 4 * CAP,), jnp.int32),
        jax.ShapeDtypeStruct((NW * 4 * CAP,), jnp.int32),
        jax.ShapeDtypeStruct((NW * 16,), jnp.int32),
    ),
    mesh=_mesh,
    scratch_types=[
        pltpu.VMEM((BSEB + L,), jnp.int32),
        pltpu.VMEM((BSEB + L,), jnp.int32),
        pltpu.VMEM((4 * SSTG + L,), jnp.int32),
        pltpu.VMEM((4 * SSTG + L,), jnp.int32),
        pltpu.VMEM((L,), jnp.int32),
    ],
)
def _sc_bin(src_hbm, dst_hbm, sfill_hbm, lfill_hbm,
            bsrc_hbm, bloc_hbm, counts_hbm,
            sstage, dstage, stgs, stgl, cntv):
    cid = lax.axis_index("c")
    sid = lax.axis_index("s")
    w = cid * NS + sid
    ebase = w * (EPAD // NW)
    hbase = w * 4 * CAP
    pltpu.sync_copy(sfill_hbm, stgs)
    pltpu.sync_copy(lfill_hbm, stgl)
    iota = lax.iota(jnp.int32, L)

    def stage_loop(st, carry):
        sb = ebase + st * BSEB
        pltpu.sync_copy(src_hbm.at[pl.ds(sb, BSEB)], sstage.at[pl.ds(0, BSEB)])
        pltpu.sync_copy(dst_hbm.at[pl.ds(sb, BSEB)], dstage.at[pl.ds(0, BSEB)])

        def edge_loop(i, cr):
            o0, o1, o2, o3, f0, f1, f2, f3 = cr
            sv = sstage[pl.ds(i, L)][0]
            dv = dstage[pl.ds(i, L)][0]
            c = ((dv >= CHUNK).astype(jnp.int32)
                 + (dv >= 2 * CHUNK).astype(jnp.int32)
                 + (dv >= 3 * CHUNK).astype(jnp.int32))
            locv = dv - c * CHUNK
            off = jnp.where(c == 0, o0,
                            jnp.where(c == 1, o1, jnp.where(c == 2, o2, o3)))
            pos = c * SSTG + off
            posa = (pos // L) * L
            lane = pos - posa
            cur = stgs[pl.ds(posa, L)]
            stgs[pl.ds(posa, L)] = jnp.where(iota == lane, sv, cur)
            curl = stgl[pl.ds(posa, L)]
            stgl[pl.ds(posa, L)] = jnp.where(iota == lane, locv, curl)
            os_ = [o0 + (c == 0), o1 + (c == 1), o2 + (c == 2), o3 + (c == 3)]
            fs_ = [f0, f1, f2, f3]
            out_o, out_f = [], []
            for b in range(4):
                flush = os_[b] == 512

                @pl.when(flush)
                def _(b=b, fb=fs_[b]):
                    fo = pl.multiple_of(hbase + b * CAP + fb, 8)
                    pltpu.sync_copy(stgs.at[pl.ds(b * SSTG, 512)],
                                    bsrc_hbm.at[pl.ds(fo, 512)])
                    pltpu.sync_copy(stgl.at[pl.ds(b * SSTG, 512)],
                                    bloc_hbm.at[pl.ds(fo, 512)])

                out_o.append(jnp.where(flush, 0, os_[b]))
                out_f.append(jnp.where(flush, fs_[b] + 512, fs_[b]))
            return tuple(out_o) + tuple(out_f)

        return lax.fori_loop(0, BSEB, edge_loop, carry)

    z = jnp.int32(0)
    carry = lax.fori_loop(0, (EPAD // NW) // BSEB, stage_loop,
                          (z, z, z, z, z, z, z, z))
    o_fin = carry[:4]
    f_fin = carry[4:]
    tots = []
    dumpv = jnp.full((L,), DUMP, jnp.int32)
    zerov = jnp.zeros((L,), jnp.int32)
    for b in range(4):
        ob, fb = o_fin[b], f_fin[b]
        oba = (ob // L) * L
        for k2 in range(5):
            pos = b * SSTG + oba + k2 * L
            keep = iota + (pos - b * SSTG) < ob
            stgs[pl.ds(pos, L)] = jnp.where(keep, stgs[pl.ds(pos, L)], zerov)
            stgl[pl.ds(pos, L)] = jnp.where(keep, stgl[pl.ds(pos, L)], dumpv)
        fo = pl.multiple_of(hbase + b * CAP + fb, 8)
        pltpu.sync_copy(stgs.at[pl.ds(b * SSTG, SSTG)],
                        bsrc_hbm.at[pl.ds(fo, SSTG)])
        pltpu.sync_copy(stgl.at[pl.ds(b * SSTG, SSTG)],
                        bloc_hbm.at[pl.ds(fo, SSTG)])
        tots.append(fb + ((ob + 63) // 64) * 64)
    cv = jnp.where(iota == 0, tots[0],
                   jnp.where(iota == 1, tots[1],
                             jnp.where(iota == 2, tots[2],
                                       jnp.where(iota == 3, tots[3], 0))))
    cntv[pl.ds(0, L)] = cv
    pltpu.sync_copy(cntv, counts_hbm.at[pl.ds(w * L, L)])


@functools.partial(
    pl.kernel,
    out_type=jax.ShapeDtypeStruct((NPAD, H), jnp.float32),
    mesh=_mesh,
    scratch_types=[
        pltpu.VMEM((EB,), jnp.int32),
        pltpu.VMEM((EB,), jnp.int32),
        pltpu.VMEM((EB,), jnp.int32),
        pltpu.VMEM((EB,), jnp.int32),
        pltpu.VMEM((EB, H), jnp.float32),
        pltpu.VMEM((EB, H), jnp.float32),
        pltpu.VMEM((2 * L,), jnp.int32),
        pltpu.VMEM_SHARED((ACC_ROWS, H), jnp.float32),
        pltpu.SemaphoreType.DMA,
        pltpu.SemaphoreType.DMA,
        pltpu.SemaphoreType.DMA,
        pltpu.SemaphoreType.DMA,
    ],
)
def _sc_agg(h_hbm, bsrc_hbm, bloc_hbm, counts_hbm, zeros_hbm, out_hbm,
            idx0, idx1, loc0, loc1, rows0, rows1, cbuf, acc,
            gsem0, gsem1, ssem0, ssem1):
    cid = lax.axis_index("c")
    sid = lax.axis_index("s")
    idxs = (idx0, idx1)
    locs = (loc0, loc1)
    rows = (rows0, rows1)
    gsems = (gsem0, gsem1)
    ssems = (ssem0, ssem1)

    for p in range(2):
        cc = cid * 2 + p
        lo = cc * CHUNK
        pltpu.sync_copy(zeros_hbm.at[pl.ds(0, ZROWS)],
                        acc.at[pl.ds(sid * ZROWS, ZROWS)])
        plsc.subcore_barrier()

        for qi in range(2):
            q = 2 * sid + qi
            pltpu.sync_copy(counts_hbm.at[pl.ds(q * L, L)],
                            cbuf.at[pl.ds(0, L)])
            cnt = cbuf[pl.ds(cc, L)][0]
            base = (q * 4 + cc) * CAP
            nblk = cnt // EB

            def blk_body(bk, _):
                slot = 0
                off = pl.multiple_of(base + bk * EB, 8)
                pltpu.sync_copy(bsrc_hbm.at[pl.ds(off, EB)], idxs[slot])
                pltpu.sync_copy(bloc_hbm.at[pl.ds(off, EB)], locs[slot])
                pltpu.async_copy(h_hbm.at[idxs[slot]], rows[slot],
                                 gsems[slot])
                pltpu.make_async_copy(h_hbm.at[idxs[slot]], rows[slot],
                                      gsems[slot]).wait()
                pltpu.sync_copy(rows[slot], acc.at[locs[slot]], add=True)
                return 0

            lax.fori_loop(0, nblk, blk_body, 0)

        plsc.subcore_barrier()
        pltpu.sync_copy(acc.at[pl.ds(sid * WB_ROWS, WB_ROWS)],
                        out_hbm.at[pl.ds(lo + sid * WB_ROWS, WB_ROWS)])
        plsc.subcore_barrier()


@functools.partial(
    pl.kernel,
    out_type=(
        jax.ShapeDtypeStruct((NC * SROWS, H), jnp.float32),
        jax.ShapeDtypeStruct((NC * SROWS, H), jnp.float32),
        jax.ShapeDtypeStruct((NW * SROWS, H), jnp.float32),
    ),
    mesh=_mesh,
    scratch_types=[
        pltpu.VMEM((PNB, H), jnp.float32),
        pltpu.VMEM((PNB,), jnp.int32),
        pltpu.VMEM((PNB + L,), jnp.int32),
        pltpu.VMEM((PNB, H), jnp.float32),
        pltpu.VMEM((SROWS, H), jnp.float32),
        pltpu.VMEM_SHARED((SROWS, H), jnp.float32),
        pltpu.VMEM_SHARED((SROWS, H), jnp.float32),
    ],
)
def _sc_pool(h_hbm, batch_hbm, zeros_hbm, ones_hbm,
             sums_hbm, cnts_hbm, maxp_hbm,
             rows, bstage, bpad, ones_v, maxacc, sumacc, cntacc):
    cid = lax.axis_index("c")
    sid = lax.axis_index("s")
    w = cid * NS + sid
    zr = SROWS // NS
    pltpu.sync_copy(zeros_hbm.at[pl.ds(0, SROWS)], maxacc)
    pltpu.sync_copy(ones_hbm, ones_v)
    pltpu.sync_copy(zeros_hbm.at[pl.ds(0, zr)], sumacc.at[pl.ds(sid * zr, zr)])
    pltpu.sync_copy(zeros_hbm.at[pl.ds(0, zr)], cntacc.at[pl.ds(sid * zr, zr)])
    plsc.subcore_barrier()

    for t in range(PN_BLKS):
        nbase = w * NODES_PER_TILE + t * PNB
        pltpu.sync_copy(h_hbm.at[pl.ds(nbase, PNB)], rows)
        pltpu.sync_copy(batch_hbm.at[pl.ds(nbase, PNB)], bstage)
        pltpu.sync_copy(batch_hbm.at[pl.ds(nbase, PNB)], bpad.at[pl.ds(0, PNB)])
        pltpu.sync_copy(rows, sumacc.at[bstage], add=True)
        pltpu.sync_copy(ones_v, cntacc.at[bstage], add=True)

        def nbody(i, _):
            b = bpad[pl.ds(i, L)][0]
            for j in range(H // L):
                sl = pl.ds(j * L, L)
                maxacc[b, sl] = jnp.maximum(maxacc[b, sl], rows[i, sl])
            return 0

        lax.fori_loop(0, PNB, nbody, 0)

    pltpu.sync_copy(maxacc, maxp_hbm.at[pl.ds(w * SROWS, SROWS)])
    plsc.subcore_barrier()
    pltpu.sync_copy(sumacc.at[pl.ds(sid * zr, zr)],
                    sums_hbm.at[pl.ds(cid * SROWS + sid * zr, zr)])
    pltpu.sync_copy(cntacc.at[pl.ds(sid * zr, zr)],
                    cnts_hbm.at[pl.ds(cid * SROWS + sid * zr, zr)])


BM = 512


def _mlp_body(h_ref, a_ref, w1_ref, b1_ref, w2_ref, b2_ref, o_ref):
    z = h_ref[...] + a_ref[...]
    t = jnp.maximum(
        jnp.dot(z, w1_ref[...], preferred_element_type=jnp.float32)
        + b1_ref[...], 0.0)
    o_ref[...] = jnp.maximum(
        jnp.dot(t, w2_ref[...], preferred_element_type=jnp.float32)
        + b2_ref[...], 0.0)


def _tc_mlp(h, agg, w1, b1, w2, b2):
    return pl.pallas_call(
        _mlp_body,
        grid=(NPAD // BM,),
        in_specs=[
            pl.BlockSpec((BM, H), lambda i: (i, 0)),
            pl.BlockSpec((BM, H), lambda i: (i, 0)),
            pl.BlockSpec((H, H), lambda i: (0, 0)),
            pl.BlockSpec((1, H), lambda i: (0, 0)),
            pl.BlockSpec((H, H), lambda i: (0, 0)),
            pl.BlockSpec((1, H), lambda i: (0, 0)),
        ],
        out_specs=pl.BlockSpec((BM, H), lambda i: (i, 0)),
        out_shape=jax.ShapeDtypeStruct((NPAD, H), jnp.float32),
    )(h, agg, w1, b1.reshape(1, H), w2, b2.reshape(1, H))


def _final_body(sum_ref, cnt_ref, maxp_ref, wf_ref, bf_ref, o_ref):
    s = sum_ref[0, :G, :] + sum_ref[1, :G, :]
    c = cnt_ref[0, :G, 0:1] + cnt_ref[1, :G, 0:1]
    mean = s / jnp.clip(c, 1.0, None)
    mx = jnp.max(maxp_ref[:, :G, :], axis=0)
    o = (jnp.dot(mean, wf_ref[:H, :], preferred_element_type=jnp.float32)
         + jnp.dot(mx, wf_ref[H:, :], preferred_element_type=jnp.float32)
         + bf_ref[...])
    o_ref[...] = o


def _tc_final(sums, cnts, maxp, wf, bf):
    return pl.pallas_call(
        _final_body,
        out_shape=jax.ShapeDtypeStruct((G, H), jnp.float32),
    )(sums, cnts, maxp, wf, bf.reshape(1, H))


def kernel(x, edge_index, batch,
           W1_0, b1_0, W2_0, b2_0,
           W1_1, b1_1, W2_1, b2_1,
           W1_2, b1_2, W2_2, b2_2,
           Wf, bf):
    src = jnp.pad(edge_index[0], (0, EPAD - E))
    dst = jnp.pad(edge_index[1], (0, EPAD - E), constant_values=N)
    xp = jnp.pad(x, ((0, NPAD - N), (0, H - F_IN)))
    w1_0p = jnp.pad(W1_0, ((0, H - F_IN), (0, 0)))
    batchp = jnp.pad(batch, (0, NPAD - N), constant_values=G)

    zeros = jnp.zeros((ACC_ROWS, H), jnp.float32)
    sfill = jnp.zeros((4 * SSTG + L,), jnp.int32)
    lfill = jnp.full((4 * SSTG + L,), DUMP, jnp.int32)
    bsrc, bloc, counts = _sc_bin(src, dst, sfill, lfill)
    ones = jnp.ones((PNB, H), jnp.float32)

    h = xp
    for w1, b1, w2, b2 in ((w1_0p, b1_0, W2_0, b2_0),
                           (W1_1, b1_1, W2_1, b2_1),
                           (W1_2, b1_2, W2_2, b2_2)):
        agg = _sc_agg(h, bsrc, bloc, counts, zeros)
        h = _tc_mlp(h, agg, w1, b1, w2, b2)

    sums, cnts, maxp = _sc_pool(h, batchp, zeros, ones)
    return _tc_final(sums.reshape(NC, SROWS, H), cnts.reshape(NC, SROWS, H),
                     maxp.reshape(NW, SROWS, H), Wf, bf)


# speedup vs baseline: 1.5844x; 1.0757x over previous
import functools

import jax
import jax.numpy as jnp
from jax import lax
from jax.experimental import pallas as pl
from jax.experimental.pallas import tpu as pltpu
from jax.experimental.pallas import tpu_sc as plsc

N = 50000
E = 800000
G = 512
H = 128
F_IN = 78

NC, NS, L = 2, 16, 16
NW = NC * NS

CHUNK = 12544
NPAD = 4 * CHUNK
DUMP = CHUNK
ACC_ROWS = 12672
ZROWS = 792

EPAD = 819200
EB = 64
SEB = 2048
EDGES_PER_TILE = EPAD // NS
N_STAGES = EDGES_PER_TILE // SEB
BLKS_PER_STAGE = SEB // EB
WB_ROWS = CHUNK // NS

SROWS = 640
PNB = 112
NODES_PER_TILE = NPAD // NW
PN_BLKS = NODES_PER_TILE // PNB

_mesh = plsc.VectorSubcoreMesh(
    core_axis_name="c", subcore_axis_name="s", num_cores=NC, num_subcores=NS)


SSTG = 576
BSEB = 1600
CAP = 25664


@functools.partial(
    pl.kernel,
    out_type=(
        jax.ShapeDtypeStruct((NW * 4 * CAP,), jnp.int32),
        jax.ShapeDtypeStruct((NW * 4 * CAP,), jnp.int32),
        jax.ShapeDtypeStruct((NW * 16,), jnp.int32),
    ),
    mesh=_mesh,
    scratch_types=[
        pltpu.VMEM((BSEB + L,), jnp.int32),
        pltpu.VMEM((BSEB + L,), jnp.int32),
        pltpu.VMEM((4 * SSTG + L,), jnp.int32),
        pltpu.VMEM((4 * SSTG + L,), jnp.int32),
        pltpu.VMEM((L,), jnp.int32),
    ],
)
def _sc_bin(src_hbm, dst_hbm, sfill_hbm, lfill_hbm,
            bsrc_hbm, bloc_hbm, counts_hbm,
            sstage, dstage, stgs, stgl, cntv):
    cid = lax.axis_index("c")
    sid = lax.axis_index("s")
    w = cid * NS + sid
    ebase = w * (EPAD // NW)
    hbase = w * 4 * CAP
    pltpu.sync_copy(sfill_hbm, stgs)
    pltpu.sync_copy(lfill_hbm, stgl)
    iota = lax.iota(jnp.int32, L)

    def stage_loop(st, carry):
        sb = ebase + st * BSEB
        pltpu.sync_copy(src_hbm.at[pl.ds(sb, BSEB)], sstage.at[pl.ds(0, BSEB)])
        pltpu.sync_copy(dst_hbm.at[pl.ds(sb, BSEB)], dstage.at[pl.ds(0, BSEB)])

        def edge_loop(i, cr):
            o0, o1, o2, o3, f0, f1, f2, f3 = cr
            sv = sstage[pl.ds(i, L)][0]
            dv = dstage[pl.ds(i, L)][0]
            c = ((dv >= CHUNK).astype(jnp.int32)
                 + (dv >= 2 * CHUNK).astype(jnp.int32)
                 + (dv >= 3 * CHUNK).astype(jnp.int32))
            locv = dv - c * CHUNK
            off = jnp.where(c == 0, o0,
                            jnp.where(c == 1, o1, jnp.where(c == 2, o2, o3)))
            pos = c * SSTG + off
            posa = (pos // L) * L
            lane = pos - posa
            cur = stgs[pl.ds(posa, L)]
            stgs[pl.ds(posa, L)] = jnp.where(iota == lane, sv, cur)
            curl = stgl[pl.ds(posa, L)]
            stgl[pl.ds(posa, L)] = jnp.where(iota == lane, locv, curl)
            os_ = [o0 + (c == 0), o1 + (c == 1), o2 + (c == 2), o3 + (c == 3)]
            fs_ = [f0, f1, f2, f3]
            out_o, out_f = [], []
            for b in range(4):
                flush = os_[b] == 512

                @pl.when(flush)
                def _(b=b, fb=fs_[b]):
                    fo = pl.multiple_of(hbase + b * CAP + fb, 8)
                    pltpu.sync_copy(stgs.at[pl.ds(b * SSTG, 512)],
                                    bsrc_hbm.at[pl.ds(fo, 512)])
                    pltpu.sync_copy(stgl.at[pl.ds(b * SSTG, 512)],
                                    bloc_hbm.at[pl.ds(fo, 512)])

                out_o.append(jnp.where(flush, 0, os_[b]))
                out_f.append(jnp.where(flush, fs_[b] + 512, fs_[b]))
            return tuple(out_o) + tuple(out_f)

        return lax.fori_loop(0, BSEB, edge_loop, carry)

    z = jnp.int32(0)
    carry = lax.fori_loop(0, (EPAD // NW) // BSEB, stage_loop,
                          (z, z, z, z, z, z, z, z))
    o_fin = carry[:4]
    f_fin = carry[4:]
    tots = []
    dumpv = jnp.full((L,), DUMP, jnp.int32)
    zerov = jnp.zeros((L,), jnp.int32)
    for b in range(4):
        ob, fb = o_fin[b], f_fin[b]
        oba = (ob // L) * L
        for k2 in range(5):
            pos = b * SSTG + oba + k2 * L
            keep = iota + (pos - b * SSTG) < ob
            stgs[pl.ds(pos, L)] = jnp.where(keep, stgs[pl.ds(pos, L)], zerov)
            stgl[pl.ds(pos, L)] = jnp.where(keep, stgl[pl.ds(pos, L)], dumpv)
        fo = pl.multiple_of(hbase + b * CAP + fb, 8)
        pltpu.sync_copy(stgs.at[pl.ds(b * SSTG, SSTG)],
                        bsrc_hbm.at[pl.ds(fo, SSTG)])
        pltpu.sync_copy(stgl.at[pl.ds(b * SSTG, SSTG)],
                        bloc_hbm.at[pl.ds(fo, SSTG)])
        tots.append(fb + ((ob + 63) // 64) * 64)
    cv = jnp.where(iota == 0, tots[0],
                   jnp.where(iota == 1, tots[1],
                             jnp.where(iota == 2, tots[2],
                                       jnp.where(iota == 3, tots[3], 0))))
    cntv[pl.ds(0, L)] = cv
    pltpu.sync_copy(cntv, counts_hbm.at[pl.ds(w * L, L)])


@functools.partial(
    pl.kernel,
    out_type=jax.ShapeDtypeStruct((NPAD, H), jnp.float32),
    mesh=_mesh,
    scratch_types=[
        pltpu.VMEM((EB,), jnp.int32),
        pltpu.VMEM((EB,), jnp.int32),
        pltpu.VMEM((EB,), jnp.int32),
        pltpu.VMEM((EB,), jnp.int32),
        pltpu.VMEM((EB, H), jnp.float32),
        pltpu.VMEM((EB, H), jnp.float32),
        pltpu.VMEM((2 * L,), jnp.int32),
        pltpu.VMEM_SHARED((ACC_ROWS, H), jnp.float32),
        pltpu.SemaphoreType.DMA,
        pltpu.SemaphoreType.DMA,
        pltpu.SemaphoreType.DMA,
        pltpu.SemaphoreType.DMA,
    ],
)
def _sc_agg(h_hbm, bsrc_hbm, bloc_hbm, counts_hbm, zeros_hbm, out_hbm,
            idx0, idx1, loc0, loc1, rows0, rows1, cbuf, acc,
            gsem0, gsem1, ssem0, ssem1):
    cid = lax.axis_index("c")
    sid = lax.axis_index("s")
    idxs = (idx0, idx1)
    locs = (loc0, loc1)
    rows = (rows0, rows1)
    gsems = (gsem0, gsem1)
    ssems = (ssem0, ssem1)

    for p in range(2):
        cc = cid * 2 + p
        lo = cc * CHUNK
        pltpu.sync_copy(zeros_hbm.at[pl.ds(0, ZROWS)],
                        acc.at[pl.ds(sid * ZROWS, ZROWS)])
        plsc.subcore_barrier()

        for qi in range(2):
            q = 2 * sid + qi
            pltpu.sync_copy(counts_hbm.at[pl.ds(q * L, L)],
                            cbuf.at[pl.ds(0, L)])
            cnt = cbuf[pl.ds(cc, L)][0]
            base = (q * 4 + cc) * CAP
            nblk = cnt // EB

            def load(slot, bk):
                off = pl.multiple_of(base + bk * EB, 8)
                pltpu.sync_copy(bsrc_hbm.at[pl.ds(off, EB)], idxs[slot])
                pltpu.sync_copy(bloc_hbm.at[pl.ds(off, EB)], locs[slot])

            def gather(slot):
                pltpu.async_copy(h_hbm.at[idxs[slot]], rows[slot],
                                 gsems[slot])

            def wait_gather(slot):
                pltpu.make_async_copy(h_hbm.at[idxs[slot]], rows[slot],
                                      gsems[slot]).wait()

            def scatter(slot):
                pltpu.async_copy(rows[slot], acc.at[locs[slot]],
                                 ssems[slot], add=True)

            def wait_scatter(slot):
                pltpu.make_async_copy(rows[slot], acc.at[locs[slot]],
                                      ssems[slot]).wait()

            @pl.when(nblk > 0)
            def _():
                load(0, 0)
                gather(0)

            def grp_body(g, _):
                wait_gather(0)
                scatter(0)

                @pl.when(2 * g + 1 < nblk)
                def _():
                    @pl.when(g > 0)
                    def _():
                        wait_scatter(1)
                    load(1, 2 * g + 1)
                    gather(1)
                    wait_gather(1)
                    scatter(1)

                @pl.when(2 * g + 2 < nblk)
                def _():
                    wait_scatter(0)
                    load(0, 2 * g + 2)
                    gather(0)
                return 0

            lax.fori_loop(0, (nblk + 1) // 2, grp_body, 0)

            @pl.when(nblk >= 1)
            def _():
                wait_scatter(0)

            @pl.when(nblk >= 2)
            def _():
                wait_scatter(1)

        plsc.subcore_barrier()
        pltpu.sync_copy(acc.at[pl.ds(sid * WB_ROWS, WB_ROWS)],
                        out_hbm.at[pl.ds(lo + sid * WB_ROWS, WB_ROWS)])
        plsc.subcore_barrier()


@functools.partial(
    pl.kernel,
    out_type=(
        jax.ShapeDtypeStruct((NC * SROWS, H), jnp.float32),
        jax.ShapeDtypeStruct((NC * SROWS, H), jnp.float32),
        jax.ShapeDtypeStruct((NW * SROWS, H), jnp.float32),
    ),
    mesh=_mesh,
    scratch_types=[
        pltpu.VMEM((PNB, H), jnp.float32),
        pltpu.VMEM((PNB,), jnp.int32),
        pltpu.VMEM((PNB + L,), jnp.int32),
        pltpu.VMEM((PNB, H), jnp.float32),
        pltpu.VMEM((SROWS, H), jnp.float32),
        pltpu.VMEM_SHARED((SROWS, H), jnp.float32),
        pltpu.VMEM_SHARED((SROWS, H), jnp.float32),
    ],
)
def _sc_pool(h_hbm, batch_hbm, zeros_hbm, ones_hbm,
             sums_hbm, cnts_hbm, maxp_hbm,
             rows, bstage, bpad, ones_v, maxacc, sumacc, cntacc):
    cid = lax.axis_index("c")
    sid = lax.axis_index("s")
    w = cid * NS + sid
    zr = SROWS // NS
    pltpu.sync_copy(zeros_hbm.at[pl.ds(0, SROWS)], maxacc)
    pltpu.sync_copy(ones_hbm, ones_v)
    pltpu.sync_copy(zeros_hbm.at[pl.ds(0, zr)], sumacc.at[pl.ds(sid * zr, zr)])
    pltpu.sync_copy(zeros_hbm.at[pl.ds(0, zr)], cntacc.at[pl.ds(sid * zr, zr)])
    plsc.subcore_barrier()

    for t in range(PN_BLKS):
        nbase = w * NODES_PER_TILE + t * PNB
        pltpu.sync_copy(h_hbm.at[pl.ds(nbase, PNB)], rows)
        pltpu.sync_copy(batch_hbm.at[pl.ds(nbase, PNB)], bstage)
        pltpu.sync_copy(batch_hbm.at[pl.ds(nbase, PNB)], bpad.at[pl.ds(0, PNB)])
        pltpu.sync_copy(rows, sumacc.at[bstage], add=True)
        pltpu.sync_copy(ones_v, cntacc.at[bstage], add=True)

        def nbody(i, _):
            b = bpad[pl.ds(i, L)][0]
            for j in range(H // L):
                sl = pl.ds(j * L, L)
                maxacc[b, sl] = jnp.maximum(maxacc[b, sl], rows[i, sl])
            return 0

        lax.fori_loop(0, PNB, nbody, 0)

    pltpu.sync_copy(maxacc, maxp_hbm.at[pl.ds(w * SROWS, SROWS)])
    plsc.subcore_barrier()
    pltpu.sync_copy(sumacc.at[pl.ds(sid * zr, zr)],
                    sums_hbm.at[pl.ds(cid * SROWS + sid * zr, zr)])
    pltpu.sync_copy(cntacc.at[pl.ds(sid * zr, zr)],
                    cnts_hbm.at[pl.ds(cid * SROWS + sid * zr, zr)])


BM = 512


def _mlp_body(h_ref, a_ref, w1_ref, b1_ref, w2_ref, b2_ref, o_ref):
    z = h_ref[...] + a_ref[...]
    t = jnp.maximum(
        jnp.dot(z, w1_ref[...], preferred_element_type=jnp.float32)
        + b1_ref[...], 0.0)
    o_ref[...] = jnp.maximum(
        jnp.dot(t, w2_ref[...], preferred_element_type=jnp.float32)
        + b2_ref[...], 0.0)


def _tc_mlp(h, agg, w1, b1, w2, b2):
    return pl.pallas_call(
        _mlp_body,
        grid=(NPAD // BM,),
        in_specs=[
            pl.BlockSpec((BM, H), lambda i: (i, 0)),
            pl.BlockSpec((BM, H), lambda i: (i, 0)),
            pl.BlockSpec((H, H), lambda i: (0, 0)),
            pl.BlockSpec((1, H), lambda i: (0, 0)),
            pl.BlockSpec((H, H), lambda i: (0, 0)),
            pl.BlockSpec((1, H), lambda i: (0, 0)),
        ],
        out_specs=pl.BlockSpec((BM, H), lambda i: (i, 0)),
        out_shape=jax.ShapeDtypeStruct((NPAD, H), jnp.float32),
    )(h, agg, w1, b1.reshape(1, H), w2, b2.reshape(1, H))


def _final_body(sum_ref, cnt_ref, maxp_ref, wf_ref, bf_ref, o_ref):
    s = sum_ref[0, :G, :] + sum_ref[1, :G, :]
    c = cnt_ref[0, :G, 0:1] + cnt_ref[1, :G, 0:1]
    mean = s / jnp.clip(c, 1.0, None)
    mx = jnp.max(maxp_ref[:, :G, :], axis=0)
    o = (jnp.dot(mean, wf_ref[:H, :], preferred_element_type=jnp.float32)
         + jnp.dot(mx, wf_ref[H:, :], preferred_element_type=jnp.float32)
         + bf_ref[...])
    o_ref[...] = o


def _tc_final(sums, cnts, maxp, wf, bf):
    return pl.pallas_call(
        _final_body,
        out_shape=jax.ShapeDtypeStruct((G, H), jnp.float32),
    )(sums, cnts, maxp, wf, bf.reshape(1, H))


def kernel(x, edge_index, batch,
           W1_0, b1_0, W2_0, b2_0,
           W1_1, b1_1, W2_1, b2_1,
           W1_2, b1_2, W2_2, b2_2,
           Wf, bf):
    src = jnp.pad(edge_index[0], (0, EPAD - E))
    dst = jnp.pad(edge_index[1], (0, EPAD - E), constant_values=N)
    xp = jnp.pad(x, ((0, NPAD - N), (0, H - F_IN)))
    w1_0p = jnp.pad(W1_0, ((0, H - F_IN), (0, 0)))
    batchp = jnp.pad(batch, (0, NPAD - N), constant_values=G)

    zeros = jnp.zeros((ACC_ROWS, H), jnp.float32)
    sfill = jnp.zeros((4 * SSTG + L,), jnp.int32)
    lfill = jnp.full((4 * SSTG + L,), DUMP, jnp.int32)
    bsrc, bloc, counts = _sc_bin(src, dst, sfill, lfill)
    ones = jnp.ones((PNB, H), jnp.float32)

    h = xp
    for w1, b1, w2, b2 in ((w1_0p, b1_0, W2_0, b2_0),
                           (W1_1, b1_1, W2_1, b2_1),
                           (W1_2, b1_2, W2_2, b2_2)):
        agg = _sc_agg(h, bsrc, bloc, counts, zeros)
        h = _tc_mlp(h, agg, w1, b1, w2, b2)

    sums, cnts, maxp = _sc_pool(h, batchp, zeros, ones)
    return _tc_final(sums.reshape(NC, SROWS, H), cnts.reshape(NC, SROWS, H),
                     maxp.reshape(NW, SROWS, H), Wf, bf)
